# baseline (speedup 1.0000x reference)
import jax
import jax.numpy as jnp
from jax.experimental import pallas as pl
from jax.experimental.pallas import tpu as pltpu

_S = 512
_SC = 128
_NJ = _S // _SC
_W = 640
_NBUF = 4
_BB = 4


def _pool_kernel(lengths_ref, stepdiv_ref, feat_hbm, mask_hbm,
                 feats_out_ref, mask_out_ref, fbuf, mbuf, fsems, msems):
    k = pl.program_id(0)
    n = pl.num_programs(0)
    T = feat_hbm.shape[-1]
    sd = stepdiv_ref[0]

    def fcopy(kk, slot):
        return pltpu.make_async_copy(feat_hbm.at[pl.ds(kk * _BB, _BB)],
                                     fbuf.at[slot], fsems.at[slot])

    def mcopy(kk, slot):
        return pltpu.make_async_copy(mask_hbm.at[pl.ds(kk * _BB, _BB)],
                                     mbuf.at[slot], msems.at[slot])

    @pl.when(k == 0)
    def _():
        for i in range(_NBUF - 1):
            fcopy(i, i).start()
            mcopy(i, i).start()

    nxt = k + _NBUF - 1
    @pl.when(nxt < n)
    def _():
        slot = jax.lax.rem(nxt, _NBUF)
        fcopy(nxt, slot).start()
        mcopy(nxt, slot).start()

    slot = jax.lax.rem(k, _NBUF)
    fcopy(k, slot).wait()
    mcopy(k, slot).wait()

    for bb in range(_BB):
      L = lengths_ref[k * _BB + bb]
      off = T - L
      for j in range(_NJ):
        lo = (L * (j * _SC)) // sd + off
        t0 = jnp.minimum((lo // 128) * 128, T - _W)

        s = jax.lax.broadcasted_iota(jnp.int32, (1, _SC), 1) + j * _SC
        start_idx = (L * s) // sd + off
        end_idx = jnp.minimum((L * (s + 1) + sd - 1) // sd + off, T)
        counts = jnp.maximum(end_idx - start_idx, 1)

        t = jax.lax.broadcasted_iota(jnp.int32, (_W, _SC), 0) + t0
        in_win = (t - start_idx).astype(jnp.uint32) < counts.astype(jnp.uint32)
        sel = jnp.where(in_win, jnp.float32(1), jnp.float32(0))

        inv = 1.0 / counts.astype(jnp.float32)
        fwin = fbuf[slot, bb, :, pl.ds(t0, _W)]
        feats_out_ref[bb, :, j * _SC:(j + 1) * _SC] = (
            jnp.dot(fwin, sel, preferred_element_type=jnp.float32) * inv)
        mwin = mbuf[slot, bb, :, pl.ds(t0, _W)]
        mask_out_ref[bb, :, j * _SC:(j + 1) * _SC] = (
            jnp.dot(mwin, sel, preferred_element_type=jnp.float32) * inv)


def kernel(features, mask, valid_lengths, target_len):
    BN, C, T = features.shape
    lengths = jnp.clip(valid_lengths.astype(jnp.int32), 1, T)
    step_div = jnp.maximum(jnp.asarray(target_len, jnp.int32), 1).reshape(1)

    grid_spec = pltpu.PrefetchScalarGridSpec(
        num_scalar_prefetch=2,
        grid=(BN // _BB,),
        in_specs=[
            pl.BlockSpec(memory_space=pl.ANY),
            pl.BlockSpec(memory_space=pl.ANY),
        ],
        out_specs=[
            pl.BlockSpec((_BB, C, _S), lambda b, *_: (b, 0, 0)),
            pl.BlockSpec((_BB, 1, _S), lambda b, *_: (b, 0, 0)),
        ],
        scratch_shapes=[
            pltpu.VMEM((_NBUF, _BB, C, T), jnp.float32),
            pltpu.VMEM((_NBUF, _BB, 1, T), jnp.float32),
            pltpu.SemaphoreType.DMA((_NBUF,)),
            pltpu.SemaphoreType.DMA((_NBUF,)),
        ],
    )
    pooled_feats, pooled_mask = pl.pallas_call(
        _pool_kernel,
        grid_spec=grid_spec,
        out_shape=[
            jax.ShapeDtypeStruct((BN, C, _S), features.dtype),
            jax.ShapeDtypeStruct((BN, 1, _S), mask.dtype),
        ],
    )(lengths, step_div, features, mask)
    return pooled_feats, pooled_mask


# speedup vs baseline: 1.0343x; 1.0343x over previous
import jax
import jax.numpy as jnp
from jax.experimental import pallas as pl
from jax.experimental.pallas import tpu as pltpu

_S = 512
_SC = 128
_NJ = _S // _SC
_W = 640
_NBUF = 4
_BB = 2


def _pool_kernel(lengths_ref, stepdiv_ref, feat_hbm, mask_hbm,
                 feats_out_ref, mask_out_ref, fbuf, mbuf, fsems, msems):
    k = pl.program_id(0)
    n = pl.num_programs(0)
    T = feat_hbm.shape[-1]
    sd = stepdiv_ref[0]

    def fcopy(kk, slot):
        return pltpu.make_async_copy(feat_hbm.at[pl.ds(kk * _BB, _BB)],
                                     fbuf.at[slot], fsems.at[slot])

    def mcopy(kk, slot):
        return pltpu.make_async_copy(mask_hbm.at[pl.ds(kk * _BB, _BB)],
                                     mbuf.at[slot], msems.at[slot])

    @pl.when(k == 0)
    def _():
        for i in range(_NBUF - 1):
            fcopy(i, i).start()
            mcopy(i, i).start()

    nxt = k + _NBUF - 1
    @pl.when(nxt < n)
    def _():
        slot = jax.lax.rem(nxt, _NBUF)
        fcopy(nxt, slot).start()
        mcopy(nxt, slot).start()

    slot = jax.lax.rem(k, _NBUF)
    fcopy(k, slot).wait()
    mcopy(k, slot).wait()

    for bb in range(_BB):
      L = lengths_ref[k * _BB + bb]
      off = T - L
      for j in range(_NJ):
        lo = (L * (j * _SC)) // sd + off
        t0 = jnp.minimum((lo // 128) * 128, T - _W)

        s = jax.lax.broadcasted_iota(jnp.int32, (1, _SC), 1) + j * _SC
        start_idx = (L * s) // sd + off
        end_idx = jnp.minimum((L * (s + 1) + sd - 1) // sd + off, T)
        counts = jnp.maximum(end_idx - start_idx, 1)

        t = jax.lax.broadcasted_iota(jnp.int32, (_W, _SC), 0) + t0
        in_win = (t - start_idx).astype(jnp.uint32) < counts.astype(jnp.uint32)
        sel = jnp.where(in_win, jnp.float32(1), jnp.float32(0))

        inv = 1.0 / counts.astype(jnp.float32)
        fwin = fbuf[slot, bb, :, pl.ds(t0, _W)]
        feats_out_ref[bb, :, j * _SC:(j + 1) * _SC] = (
            jnp.dot(fwin, sel, preferred_element_type=jnp.float32) * inv)
        mwin = mbuf[slot, bb, :, pl.ds(t0, _W)]
        mask_out_ref[bb, :, j * _SC:(j + 1) * _SC] = (
            jnp.dot(mwin, sel, preferred_element_type=jnp.float32) * inv)


def kernel(features, mask, valid_lengths, target_len):
    BN, C, T = features.shape
    lengths = jnp.clip(valid_lengths.astype(jnp.int32), 1, T)
    step_div = jnp.maximum(jnp.asarray(target_len, jnp.int32), 1).reshape(1)

    grid_spec = pltpu.PrefetchScalarGridSpec(
        num_scalar_prefetch=2,
        grid=(BN // _BB,),
        in_specs=[
            pl.BlockSpec(memory_space=pl.ANY),
            pl.BlockSpec(memory_space=pl.ANY),
        ],
        out_specs=[
            pl.BlockSpec((_BB, C, _S), lambda b, *_: (b, 0, 0)),
            pl.BlockSpec((_BB, 1, _S), lambda b, *_: (b, 0, 0)),
        ],
        scratch_shapes=[
            pltpu.VMEM((_NBUF, _BB, C, T), jnp.float32),
            pltpu.VMEM((_NBUF, _BB, 1, T), jnp.float32),
            pltpu.SemaphoreType.DMA((_NBUF,)),
            pltpu.SemaphoreType.DMA((_NBUF,)),
        ],
    )
    pooled_feats, pooled_mask = pl.pallas_call(
        _pool_kernel,
        grid_spec=grid_spec,
        out_shape=[
            jax.ShapeDtypeStruct((BN, C, _S), features.dtype),
            jax.ShapeDtypeStruct((BN, 1, _S), mask.dtype),
        ],
    )(lengths, step_div, features, mask)
    return pooled_feats, pooled_mask
